# Initial kernel scaffold; baseline (speedup 1.0000x reference)
import jax
import jax.numpy as jnp
from jax.experimental import pallas as pl


def kernel(x, w1, g1, b1, m1, v1, se_w, se_b, w_dw, g2, b2, m2, v2, w2, g3, b3, m3, v3):
    raise NotImplementedError("write your pallas kernel here")


# speedup vs baseline: 10.7278x; 10.7278x over previous
import jax
import jax.numpy as jnp
from jax.experimental import pallas as pl
from jax.experimental.pallas import tpu as pltpu

_B, _INP, _OUP, _HID, _H, _W = 16, 32, 64, 192, 96, 96
_TC, _TR = _W // 2, _H // 2

_HIGH = jax.lax.Precision.HIGHEST
_F32 = jnp.float32


def _fiota(shape, dim):
    return jax.lax.broadcasted_iota(jnp.int32, shape, dim).astype(_F32)


def _body(xt_ref, w1_ref, b1_ref, sw_ref, seb_ref, wdw_ref, b2_ref,
          w2_ref, b3_ref, out_ref):
    H, W, HID, TC, TR = _H, _W, _HID, _TC, _TR

    x2 = xt_ref[0]
    y2 = jnp.dot(x2, w1_ref[...], preferred_element_type=_F32)
    y2 = jnp.clip(y2 + b1_ref[...], 0.0, 6.0)
    y = y2.reshape(H, W, HID)

    q = jnp.sum(y * sw_ref[...][None], axis=-1)
    pi = _fiota((6, H), 0)
    ph = _fiota((6, H), 1)
    P = jnp.where(jnp.floor(ph / 16.0) == pi, 1.0 / 16.0, 0.0)
    qp = jnp.dot(jnp.dot(P, q, precision=_HIGH, preferred_element_type=_F32),
                 P.T, precision=_HIGH, preferred_element_type=_F32)
    se6 = jax.nn.sigmoid(qp + seb_ref[...])
    ur = _fiota((H, 6), 0)
    uj = _fiota((H, 6), 1)
    c = ur * (5.0 / 95.0)
    i0 = jnp.clip(jnp.floor(c), 0.0, 4.0)
    fr = c - i0
    U = jnp.where(uj == i0, 1.0 - fr, 0.0) + jnp.where(uj == i0 + 1.0, fr, 0.0)
    se = jnp.dot(jnp.dot(U, se6, precision=_HIGH, preferred_element_type=_F32),
                 U.T, precision=_HIGH, preferred_element_type=_F32)

    rm = se + 0.001
    rowsum = jnp.sum(rm, axis=1, keepdims=True)
    a = (W * 0.5) / rowsum * rm + 0.5
    tio_r = _fiota((W, W), 0)
    tio_c = _fiota((W, W), 1)
    LT = (tio_r <= tio_c).astype(_F32)
    xx = jnp.dot(a, LT, precision=_HIGH, preferred_element_type=_F32)

    cq = _fiota((1, TC), 1) * 2.0
    xx3 = xx[:, None, :]
    cnt = jnp.sum((xx3 < cq[:, :, None]).astype(_F32), axis=-1)
    ifl = jnp.clip(cnt - 1.0, 0.0, W - 2.0)
    wio = _fiota((H, TC, W), 2)
    oh0 = (wio == ifl[:, :, None]).astype(_F32)
    oh1 = (wio == ifl[:, :, None] + 1.0).astype(_F32)
    x0 = jnp.sum(oh0 * xx3, axis=-1)
    x1 = jnp.sum(oh1 * xx3, axis=-1)
    t = (cq - x0) / (x1 - x0)
    Rt = oh0 * (1.0 - t)[:, :, None] + oh1 * t[:, :, None]
    Rse = Rt * se[:, None, :]
    mm = jnp.sum(Rse, axis=-1)
    mf = jax.lax.dot_general(Rse, y, (((2,), (1,)), ((0,), (0,))),
                             preferred_element_type=_F32)

    cm = mm + 0.001
    colsum = jnp.sum(cm, axis=0, keepdims=True)
    acol = (H * 0.5) / colsum * cm + 0.5
    LT2 = (tio_r >= tio_c).astype(_F32)
    yy = jnp.dot(LT2, acol, precision=_HIGH, preferred_element_type=_F32)
    yyT = yy.T
    rq = _fiota((1, TR), 1) * 2.0
    yy3 = yyT[:, None, :]
    cnt2 = jnp.sum((yy3 < rq[:, :, None]).astype(_F32), axis=-1)
    i2 = jnp.clip(cnt2 - 1.0, 0.0, H - 2.0)
    hio = _fiota((TC, TR, H), 2)
    oha = (hio == i2[:, :, None]).astype(_F32)
    ohb = (hio == i2[:, :, None] + 1.0).astype(_F32)
    p0 = jnp.sum(oha * yy3, axis=-1)
    p1 = jnp.sum(ohb * yy3, axis=-1)
    t2 = (rq - p0) / (p1 - p0)
    S = oha * (1.0 - t2)[:, :, None] + ohb * t2[:, :, None]
    ff = jax.lax.dot_general(S, mf, (((2,), (0,)), ((0,), (1,))),
                             preferred_element_type=_F32)

    zj = jnp.zeros((1, TR, HID), _F32)
    fp = jnp.concatenate([zj, ff, zj], axis=0)
    zk = jnp.zeros((TC + 2, 1, HID), _F32)
    fp = jnp.concatenate([zk, fp, zk], axis=1)
    acc = b2_ref[...][None]
    for kh in range(3):
        for kw in range(3):
            tap = wdw_ref[kh * 3 + kw:kh * 3 + kw + 1, :][None]
            acc = acc + fp[kw:kw + TC, kh:kh + TR, :] * tap
    z = jnp.clip(acc, 0.0, 6.0)

    zp = jnp.dot(z.reshape(TC * TR, HID), w2_ref[...],
                 preferred_element_type=_F32) + b3_ref[...]
    out_ref[0] = zp.reshape(TC, TR, _OUP)


def kernel(x, w1, g1, b1, m1, v1, se_w, se_b, w_dw, g2, b2, m2, v2,
           w2, g3, b3, m3, v3):
    s1 = g1 / jnp.sqrt(v1 + 1e-5)
    w1f = (w1 * s1[:, None]).T
    b1f = (b1 - m1 * s1)[None, :]
    seb = se_b.reshape(1, 1)
    s2 = g2 / jnp.sqrt(v2 + 1e-5)
    wdw9 = (w_dw[:, 0] * s2[:, None, None]).transpose(1, 2, 0).reshape(9, _HID)
    b2f = (b2 - m2 * s2)[None, :]
    s3 = g3 / jnp.sqrt(v3 + 1e-5)
    w2f = (w2 * s3[:, None]).T
    b3f = (b3 - m3 * s3)[None, :]
    xt = x.transpose(0, 2, 3, 1).reshape(_B, _H * _W, _INP)

    full = lambda s: pl.BlockSpec(s, lambda b: (0,) * len(s))
    out = pl.pallas_call(
        _body,
        grid=(_B,),
        in_specs=[
            pl.BlockSpec((1, _H * _W, _INP), lambda b: (b, 0, 0)),
            full(w1f.shape), full(b1f.shape), full(se_w.shape), full(seb.shape),
            full(wdw9.shape), full(b2f.shape), full(w2f.shape), full(b3f.shape),
        ],
        out_specs=pl.BlockSpec((1, _TC, _TR, _OUP), lambda b: (b, 0, 0, 0)),
        out_shape=jax.ShapeDtypeStruct((_B, _TC, _TR, _OUP), jnp.float32),
        compiler_params=pltpu.CompilerParams(dimension_semantics=("parallel",)),
    )(xt, w1f, b1f, se_w, seb, wdw9, b2f, w2f, b3f)
    return out.transpose(0, 3, 2, 1)


# speedup vs baseline: 11.2288x; 1.0467x over previous
import jax
import jax.numpy as jnp
from jax.experimental import pallas as pl
from jax.experimental.pallas import tpu as pltpu

_B, _INP, _OUP, _HID, _H, _W = 16, 32, 64, 192, 96, 96
_TC, _TR = _W // 2, _H // 2

_HIGH = jax.lax.Precision.HIGHEST
_F32 = jnp.float32


def _fiota(shape, dim):
    return jax.lax.broadcasted_iota(jnp.int32, shape, dim).astype(_F32)


def _interp_onehots(knots, nq):
    W = knots.shape[-1]
    q3 = _fiota((1, nq), 1)[:, :, None] * 2.0
    k3 = knots[:, None, :]
    nxt = jnp.concatenate([knots[:, 1:], knots[:, -1:] + 1.0], axis=1)
    lt = k3 < q3
    ge = nxt[:, None, :] >= q3
    isw0 = _fiota((1, W), 1)[:, None, :] == 0.0
    oh0 = jnp.where(jnp.logical_or(lt, isw0) & ge, 1.0, 0.0)
    oh1 = jnp.concatenate([jnp.zeros_like(oh0[..., :1]), oh0[..., :-1]],
                          axis=-1)
    x0 = jnp.sum(oh0 * k3, axis=-1)
    x1 = jnp.sum(oh1 * k3, axis=-1)
    t = (q3[:, :, 0] - x0) / (x1 - x0)
    return oh0, oh1, t


def _body(x_ref, w1_ref, b1_ref, sw_ref, seb_ref, wdw_ref, b2_ref,
          w2_ref, b3_ref, out_ref):
    H, W, HID, TC, TR = _H, _W, _HID, _TC, _TR

    y = jax.lax.dot_general(x_ref[0], w1_ref[...], (((0,), (0,)), ((), ())),
                            preferred_element_type=_F32)
    y = jnp.clip(y + b1_ref[...][None], 0.0, 6.0)

    q = jnp.sum(y * sw_ref[...][None], axis=-1)
    pi = _fiota((6, H), 0)
    ph = _fiota((6, H), 1)
    P = jnp.where(jnp.floor(ph / 16.0) == pi, 1.0 / 16.0, 0.0)
    qp = jnp.dot(jnp.dot(P, q, precision=_HIGH, preferred_element_type=_F32),
                 P.T, precision=_HIGH, preferred_element_type=_F32)
    se6 = jax.nn.sigmoid(qp + seb_ref[...])
    ur = _fiota((H, 6), 0)
    uj = _fiota((H, 6), 1)
    c = ur * (5.0 / 95.0)
    i0 = jnp.clip(jnp.floor(c), 0.0, 4.0)
    fr = c - i0
    U = jnp.where(uj == i0, 1.0 - fr, 0.0) + jnp.where(uj == i0 + 1.0, fr, 0.0)
    se = jnp.dot(jnp.dot(U, se6, precision=_HIGH, preferred_element_type=_F32),
                 U.T, precision=_HIGH, preferred_element_type=_F32)

    rm = se + 0.001
    rowsum = jnp.sum(rm, axis=1, keepdims=True)
    a = (W * 0.5) / rowsum * rm + 0.5
    tio_r = _fiota((W, W), 0)
    tio_c = _fiota((W, W), 1)
    LT = (tio_r <= tio_c).astype(_F32)
    xx = jnp.dot(a, LT, precision=_HIGH, preferred_element_type=_F32)

    oh0, oh1, t = _interp_onehots(xx, TC)
    Rt = oh0 * (1.0 - t)[:, :, None] + oh1 * t[:, :, None]
    Rse = Rt * se[:, None, :]
    mm = jnp.sum(Rse, axis=-1)
    mf = jax.lax.dot_general(Rse, y, (((2,), (1,)), ((0,), (0,))),
                             preferred_element_type=_F32)

    cm = mm + 0.001
    colsum = jnp.sum(cm, axis=0, keepdims=True)
    acol = (H * 0.5) / colsum * cm + 0.5
    LT2 = (tio_r >= tio_c).astype(_F32)
    yy = jnp.dot(LT2, acol, precision=_HIGH, preferred_element_type=_F32)
    yyT = yy.T
    oha, ohb, t2 = _interp_onehots(yyT, TR)
    S = oha * (1.0 - t2)[:, :, None] + ohb * t2[:, :, None]
    ff = jax.lax.dot_general(S, mf, (((2,), (0,)), ((0,), (1,))),
                             preferred_element_type=_F32)

    zj = jnp.zeros((1, TR, HID), _F32)
    fp = jnp.concatenate([zj, ff, zj], axis=0)
    cs = []
    for kh in range(3):
        ck = None
        for kw in range(3):
            tap = wdw_ref[kh * 3 + kw:kh * 3 + kw + 1, :][None]
            term = fp[kw:kw + TC, :, :] * tap
            ck = term if ck is None else ck + term
        cs.append(ck)
    zk = jnp.zeros((TC, 1, HID), _F32)
    acc = (cs[1]
           + jnp.concatenate([zk, cs[0][:, :-1, :]], axis=1)
           + jnp.concatenate([cs[2][:, 1:, :], zk], axis=1))
    z = jnp.clip(acc + b2_ref[...][None], 0.0, 6.0)

    zp = jnp.dot(z.reshape(TC * TR, HID), w2_ref[...],
                 preferred_element_type=_F32) + b3_ref[...]
    out_ref[0] = zp.reshape(TC, TR, _OUP)


def kernel(x, w1, g1, b1, m1, v1, se_w, se_b, w_dw, g2, b2, m2, v2,
           w2, g3, b3, m3, v3):
    s1 = g1 / jnp.sqrt(v1 + 1e-5)
    w1f = (w1 * s1[:, None]).T
    b1f = (b1 - m1 * s1)[None, :]
    seb = se_b.reshape(1, 1)
    s2 = g2 / jnp.sqrt(v2 + 1e-5)
    wdw9 = (w_dw[:, 0] * s2[:, None, None]).transpose(1, 2, 0).reshape(9, _HID)
    b2f = (b2 - m2 * s2)[None, :]
    s3 = g3 / jnp.sqrt(v3 + 1e-5)
    w2f = (w2 * s3[:, None]).T
    b3f = (b3 - m3 * s3)[None, :]

    full = lambda s: pl.BlockSpec(s, lambda b: (0,) * len(s))
    out = pl.pallas_call(
        _body,
        grid=(_B,),
        in_specs=[
            pl.BlockSpec((1, _INP, _H, _W), lambda b: (b, 0, 0, 0)),
            full(w1f.shape), full(b1f.shape), full(se_w.shape), full(seb.shape),
            full(wdw9.shape), full(b2f.shape), full(w2f.shape), full(b3f.shape),
        ],
        out_specs=pl.BlockSpec((1, _TC, _TR, _OUP), lambda b: (b, 0, 0, 0)),
        out_shape=jax.ShapeDtypeStruct((_B, _TC, _TR, _OUP), jnp.float32),
        compiler_params=pltpu.CompilerParams(dimension_semantics=("parallel",)),
    )(x, w1f, b1f, se_w, seb, wdw9, b2f, w2f, b3f)
    return out.transpose(0, 3, 2, 1)


# speedup vs baseline: 13.9001x; 1.2379x over previous
import jax
import jax.numpy as jnp
from jax.experimental import pallas as pl
from jax.experimental.pallas import tpu as pltpu

_B, _INP, _OUP, _HID, _H, _W = 16, 32, 64, 192, 96, 96
_TC, _TR = _W // 2, _H // 2

_HIGH = jax.lax.Precision.HIGHEST
_F32 = jnp.float32


def _fiota(shape, dim):
    return jax.lax.broadcasted_iota(jnp.int32, shape, dim).astype(_F32)


def _interp_matrix(knots, nq):
    W = knots.shape[-1]
    q3 = _fiota((1, nq), 1)[:, :, None] * 2.0
    k3 = knots[:, None, :]
    d = jnp.concatenate([knots[:, 1:] - knots[:, :-1],
                         jnp.ones_like(knots[:, :1])], axis=1)
    rd3 = (1.0 / d)[:, None, :]
    A = jnp.minimum((q3 - k3) * rd3, 1.0)
    isw0 = _fiota((1, W), 1)[:, None, :] == 0.0
    A = jnp.where(isw0, A, jnp.maximum(A, 0.0))
    Ashift = jnp.concatenate([jnp.ones_like(A[..., :1]), A[..., :-1]],
                             axis=-1)
    return Ashift - A


def _body(x_ref, w1_ref, b1_ref, sw_ref, seb_ref, wdw_ref, b2_ref,
          w2_ref, b3_ref, ltu_ref, ltl_ref, out_ref):
    H, W, HID, TC, TR = _H, _W, _HID, _TC, _TR

    y = jax.lax.dot_general(x_ref[0], w1_ref[...], (((0,), (0,)), ((), ())),
                            preferred_element_type=_F32)
    y = jnp.clip(y + b1_ref[...][None], 0.0, 6.0)

    q = jnp.sum(y * sw_ref[...][None], axis=-1)
    pi = _fiota((6, H), 0)
    ph = _fiota((6, H), 1)
    P = jnp.where(jnp.floor(ph / 16.0) == pi, 1.0 / 16.0, 0.0)
    qp = jnp.dot(jnp.dot(P, q, precision=_HIGH, preferred_element_type=_F32),
                 P.T, precision=_HIGH, preferred_element_type=_F32)
    se6 = jax.nn.sigmoid(qp + seb_ref[...])
    ur = _fiota((H, 6), 0)
    uj = _fiota((H, 6), 1)
    c = ur * (5.0 / 95.0)
    i0 = jnp.clip(jnp.floor(c), 0.0, 4.0)
    fr = c - i0
    U = jnp.where(uj == i0, 1.0 - fr, 0.0) + jnp.where(uj == i0 + 1.0, fr, 0.0)
    se = jnp.dot(jnp.dot(U, se6, precision=_HIGH, preferred_element_type=_F32),
                 U.T, precision=_HIGH, preferred_element_type=_F32)

    rm = se + 0.001
    rowsum = jnp.sum(rm, axis=1, keepdims=True)
    a = (W * 0.5) / rowsum * rm + 0.5
    xx = jnp.dot(a, ltu_ref[...], precision=_HIGH,
                 preferred_element_type=_F32)

    Rse = _interp_matrix(xx, TC) * se[:, None, :]
    mm = jnp.sum(Rse, axis=-1)
    mf = jax.lax.dot_general(Rse, y, (((2,), (1,)), ((0,), (0,))),
                             preferred_element_type=_F32)

    cm = mm + 0.001
    colsum = jnp.sum(cm, axis=0, keepdims=True)
    acol = (H * 0.5) / colsum * cm + 0.5
    yy = jnp.dot(ltl_ref[...], acol, precision=_HIGH,
                 preferred_element_type=_F32)
    yyT = yy.T
    S = _interp_matrix(yyT, TR)
    ff = jax.lax.dot_general(S, mf, (((2,), (0,)), ((0,), (1,))),
                             preferred_element_type=_F32)

    zj = jnp.zeros((1, TR, HID), _F32)
    fp = jnp.concatenate([zj, ff, zj], axis=0)
    cs = []
    for kh in range(3):
        ck = None
        for kw in range(3):
            tap = wdw_ref[kh * 3 + kw:kh * 3 + kw + 1, :][None]
            term = fp[kw:kw + TC, :, :] * tap
            ck = term if ck is None else ck + term
        cs.append(ck)
    zk = jnp.zeros((TC, 1, HID), _F32)
    acc = (cs[1]
           + jnp.concatenate([zk, cs[0][:, :-1, :]], axis=1)
           + jnp.concatenate([cs[2][:, 1:, :], zk], axis=1))
    z = jnp.clip(acc + b2_ref[...][None], 0.0, 6.0)

    zp = jnp.dot(z.reshape(TC * TR, HID), w2_ref[...],
                 preferred_element_type=_F32) + b3_ref[...]
    out_ref[0] = zp.reshape(TC, TR, _OUP)


def kernel(x, w1, g1, b1, m1, v1, se_w, se_b, w_dw, g2, b2, m2, v2,
           w2, g3, b3, m3, v3):
    s1 = g1 / jnp.sqrt(v1 + 1e-5)
    w1f = (w1 * s1[:, None]).T
    b1f = (b1 - m1 * s1)[None, :]
    seb = se_b.reshape(1, 1)
    s2 = g2 / jnp.sqrt(v2 + 1e-5)
    wdw9 = (w_dw[:, 0] * s2[:, None, None]).transpose(1, 2, 0).reshape(9, _HID)
    b2f = (b2 - m2 * s2)[None, :]
    s3 = g3 / jnp.sqrt(v3 + 1e-5)
    w2f = (w2 * s3[:, None]).T
    b3f = (b3 - m3 * s3)[None, :]

    wio = jnp.arange(_W, dtype=jnp.float32)
    ltu = (wio[:, None] <= wio[None, :]).astype(jnp.float32)
    ltl = ltu.T

    full = lambda s: pl.BlockSpec(s, lambda b: (0,) * len(s))
    out = pl.pallas_call(
        _body,
        grid=(_B,),
        in_specs=[
            pl.BlockSpec((1, _INP, _H, _W), lambda b: (b, 0, 0, 0)),
            full(w1f.shape), full(b1f.shape), full(se_w.shape), full(seb.shape),
            full(wdw9.shape), full(b2f.shape), full(w2f.shape), full(b3f.shape),
            full(ltu.shape), full(ltl.shape),
        ],
        out_specs=pl.BlockSpec((1, _TC, _TR, _OUP), lambda b: (b, 0, 0, 0)),
        out_shape=jax.ShapeDtypeStruct((_B, _TC, _TR, _OUP), jnp.float32),
        compiler_params=pltpu.CompilerParams(
            dimension_semantics=("arbitrary",)),
    )(x, w1f, b1f, se_w, seb, wdw9, b2f, w2f, b3f, ltu, ltl)
    return out.transpose(0, 3, 2, 1)


# speedup vs baseline: 14.8523x; 1.0685x over previous
import jax
import jax.numpy as jnp
from jax.experimental import pallas as pl
from jax.experimental.pallas import tpu as pltpu

_B, _INP, _OUP, _HID, _H, _W = 16, 32, 64, 192, 96, 96
_TC, _TR = _W // 2, _H // 2

_HIGH = jax.lax.Precision.HIGHEST
_F32 = jnp.float32


def _fiota(shape, dim):
    return jax.lax.broadcasted_iota(jnp.int32, shape, dim).astype(_F32)


def _interp_matrix(knots, nq):
    W = knots.shape[-1]
    q3 = _fiota((1, nq), 1)[:, :, None] * 2.0
    k3 = knots[:, None, :]
    d = jnp.concatenate([knots[:, 1:] - knots[:, :-1],
                         jnp.ones_like(knots[:, :1])], axis=1)
    rd3 = (1.0 / d)[:, None, :]
    A = jnp.minimum((q3 - k3) * rd3, 1.0)
    isw0 = _fiota((1, W), 1)[:, None, :] == 0.0
    A = jnp.where(isw0, A, jnp.maximum(A, 0.0))
    Ashift = jnp.concatenate([jnp.ones_like(A[..., :1]), A[..., :-1]],
                             axis=-1)
    return Ashift - A


def _body(x_ref, w1_ref, b1_ref, sw_ref, seb_ref, wdw_ref, b2_ref,
          w2_ref, b3_ref, ltu_ref, ltl_ref, out_ref):
    H, W, HID, TC, TR = _H, _W, _HID, _TC, _TR

    y = jax.lax.dot_general(x_ref[0], w1_ref[...], (((0,), (0,)), ((), ())),
                            preferred_element_type=_F32)
    y = jnp.clip(y + b1_ref[...][None], 0.0, 6.0)

    q = jnp.sum(y * sw_ref[...][None], axis=-1)
    pi = _fiota((6, H), 0)
    ph = _fiota((6, H), 1)
    P = jnp.where(jnp.floor(ph / 16.0) == pi, 1.0 / 16.0, 0.0)
    qp = jnp.dot(jnp.dot(P, q, precision=_HIGH, preferred_element_type=_F32),
                 P.T, precision=_HIGH, preferred_element_type=_F32)
    se6 = jax.nn.sigmoid(qp + seb_ref[...])
    ur = _fiota((H, 6), 0)
    uj = _fiota((H, 6), 1)
    c = ur * (5.0 / 95.0)
    i0 = jnp.clip(jnp.floor(c), 0.0, 4.0)
    fr = c - i0
    U = jnp.where(uj == i0, 1.0 - fr, 0.0) + jnp.where(uj == i0 + 1.0, fr, 0.0)
    se = jnp.dot(jnp.dot(U, se6, precision=_HIGH, preferred_element_type=_F32),
                 U.T, precision=_HIGH, preferred_element_type=_F32)

    rm = se + 0.001
    rowsum = jnp.sum(rm, axis=1, keepdims=True)
    a = (W * 0.5) / rowsum * rm + 0.5
    xx = jnp.dot(a, ltu_ref[...], precision=_HIGH,
                 preferred_element_type=_F32)

    Rse = _interp_matrix(xx, TC) * se[:, None, :]
    mm = jnp.sum(Rse, axis=-1)
    mf = jax.lax.dot_general(Rse, y, (((2,), (1,)), ((0,), (0,))),
                             preferred_element_type=_F32)

    cm = mm + 0.001
    colsum = jnp.sum(cm, axis=0, keepdims=True)
    acol = (H * 0.5) / colsum * cm + 0.5
    yy = jnp.dot(ltl_ref[...], acol, precision=_HIGH,
                 preferred_element_type=_F32)
    yyT = yy.T
    S = _interp_matrix(yyT, TR)
    ff = jax.lax.dot_general(S, mf, (((2,), (0,)), ((0,), (1,))),
                             preferred_element_type=_F32)

    zj = jnp.zeros((1, TR, HID), _F32)
    fp = jnp.concatenate([zj, ff, zj], axis=0)
    cs = []
    for kh in range(3):
        ck = None
        for kw in range(3):
            tap = wdw_ref[kh * 3 + kw:kh * 3 + kw + 1, :][None]
            term = fp[kw:kw + TC, :, :] * tap
            ck = term if ck is None else ck + term
        cs.append(ck)
    zk = jnp.zeros((TC, 1, HID), _F32)
    acc = (cs[1]
           + jnp.concatenate([zk, cs[0][:, :-1, :]], axis=1)
           + jnp.concatenate([cs[2][:, 1:, :], zk], axis=1))
    z = jnp.clip(acc + b2_ref[...][None], 0.0, 6.0)

    zp = jnp.dot(z.reshape(TC * TR, HID), w2_ref[...],
                 preferred_element_type=_F32) + b3_ref[...]
    zp3 = zp.reshape(TC, TR, _OUP)
    out_ref[0] = jnp.transpose(zp3, (2, 1, 0))


def kernel(x, w1, g1, b1, m1, v1, se_w, se_b, w_dw, g2, b2, m2, v2,
           w2, g3, b3, m3, v3):
    s1 = g1 / jnp.sqrt(v1 + 1e-5)
    w1f = (w1 * s1[:, None]).T
    b1f = (b1 - m1 * s1)[None, :]
    seb = se_b.reshape(1, 1)
    s2 = g2 / jnp.sqrt(v2 + 1e-5)
    wdw9 = (w_dw[:, 0] * s2[:, None, None]).transpose(1, 2, 0).reshape(9, _HID)
    b2f = (b2 - m2 * s2)[None, :]
    s3 = g3 / jnp.sqrt(v3 + 1e-5)
    w2f = (w2 * s3[:, None]).T
    b3f = (b3 - m3 * s3)[None, :]

    wio = jnp.arange(_W, dtype=jnp.float32)
    ltu = (wio[:, None] <= wio[None, :]).astype(jnp.float32)
    ltl = ltu.T

    full = lambda s: pl.BlockSpec(s, lambda b: (0,) * len(s))
    out = pl.pallas_call(
        _body,
        grid=(_B,),
        in_specs=[
            pl.BlockSpec((1, _INP, _H, _W), lambda b: (b, 0, 0, 0)),
            full(w1f.shape), full(b1f.shape), full(se_w.shape), full(seb.shape),
            full(wdw9.shape), full(b2f.shape), full(w2f.shape), full(b3f.shape),
            full(ltu.shape), full(ltl.shape),
        ],
        out_specs=pl.BlockSpec((1, _OUP, _TR, _TC), lambda b: (b, 0, 0, 0)),
        out_shape=jax.ShapeDtypeStruct((_B, _OUP, _TR, _TC), jnp.float32),
        compiler_params=pltpu.CompilerParams(
            dimension_semantics=("arbitrary",)),
    )(x, w1f, b1f, se_w, seb, wdw9, b2f, w2f, b3f, ltu, ltl)
    return out
